# Initial kernel scaffold; baseline (speedup 1.0000x reference)
import jax
import jax.numpy as jnp
from jax.experimental import pallas as pl


def kernel(x):
    raise NotImplementedError("write your pallas kernel here")


# speedup vs baseline: 5.5513x; 5.5513x over previous
import jax
import jax.numpy as jnp
import numpy as np
from jax import lax
from jax.experimental import pallas as pl
from jax.experimental.pallas import tpu as pltpu
from jax.experimental.pallas import tpu_sc as plsc

N_FIELDS = 26
KEYS_PER_FIELD = 16
BATCH = 16384
TOTAL = BATCH * N_FIELDS
LANES = 16
PERIOD_VECS = 13
PERIOD = PERIOD_VECS * LANES
NUM_CORES = 2
NUM_SUBCORES = 16
NUM_WORKERS = NUM_CORES * NUM_SUBCORES
CHUNK = TOTAL // NUM_WORKERS
CHUNK_VECS = CHUNK // LANES
GROUPS = CHUNK_VECS // PERIOD_VECS


NBUF = 2
HALF = CHUNK // NBUF
HALF_VECS = HALF // LANES
HALF_GROUPS = HALF_VECS // PERIOD_VECS


def _body(x_hbm, out_hbm, x_v, out_v, in_sems, out_sems):
    wid = lax.axis_index("s") * NUM_CORES + lax.axis_index("c")
    base = wid * CHUNK
    copies = []
    for h in range(NBUF):
        copies.append(pltpu.async_copy(
            x_hbm.at[pl.ds(base + h * HALF, HALF)],
            x_v.at[pl.ds(h * HALF, HALF)],
            in_sems[h],
        ))
    lane = lax.iota(jnp.int32, LANES)
    offs = [
        ((lane + (j * LANES)) % N_FIELDS) * KEYS_PER_FIELD
        for j in range(PERIOD_VECS)
    ]
    minus1 = jnp.full((LANES,), -1, dtype=jnp.int32)

    out_copies = []
    for h in range(NBUF):
        copies[h].wait()

        def group(g, carry, hbase=h * HALF):
            gbase = hbase + g * PERIOD
            for j in range(PERIOD_VECS):
                sl = pl.ds(gbase + j * LANES, LANES)
                xv = x_v[sl]
                ok = xv.astype(jnp.uint32) < KEYS_PER_FIELD
                out_v[sl] = jnp.where(ok, xv + offs[j], minus1)
            return carry

        lax.fori_loop(0, HALF_GROUPS, group, 0)
        out_copies.append(pltpu.async_copy(
            out_v.at[pl.ds(h * HALF, HALF)],
            out_hbm.at[pl.ds(base + h * HALF, HALF)],
            out_sems[h],
        ))
    for c in out_copies:
        c.wait()


@jax.jit
def kernel(x):
    run = pl.kernel(
        _body,
        out_type=jax.ShapeDtypeStruct((TOTAL,), jnp.int32),
        mesh=plsc.VectorSubcoreMesh(
            core_axis_name="c", subcore_axis_name="s",
            num_cores=NUM_CORES, num_subcores=NUM_SUBCORES,
        ),
        scratch_types=[
            pltpu.VMEM((CHUNK,), jnp.int32),
            pltpu.VMEM((CHUNK,), jnp.int32),
            [pltpu.SemaphoreType.DMA] * NBUF,
            [pltpu.SemaphoreType.DMA] * NBUF,
        ],
    )
    out = run(x.reshape(TOTAL))
    return out.reshape(BATCH, N_FIELDS)


# speedup vs baseline: 13.4196x; 2.4174x over previous
import jax
import jax.numpy as jnp
import numpy as np
from jax import lax
from jax.experimental import pallas as pl
from jax.experimental.pallas import tpu as pltpu
from jax.experimental.pallas import tpu_sc as plsc

N_FIELDS = 26
KEYS_PER_FIELD = 16
BATCH = 16384
LANES = 16
NUM_CORES = 2
NUM_SUBCORES = 16
NUM_WORKERS = NUM_CORES * NUM_SUBCORES
BCOLS = BATCH // NUM_WORKERS
CVECS = BCOLS // LANES


def _body(xt_hbm, out_hbm, x_v, out_v):
    wid = lax.axis_index("s") * NUM_CORES + lax.axis_index("c")
    base = wid * BCOLS
    pltpu.sync_copy(xt_hbm.at[:, pl.ds(base, BCOLS)], x_v)
    minus1 = jnp.full((LANES,), -1, dtype=jnp.int32)

    def group(g, carry):
        c0 = g * LANES
        for i in range(N_FIELDS):
            sl = pl.ds(c0, LANES)
            xv = x_v[i, sl]
            ok = xv.astype(jnp.uint32) < KEYS_PER_FIELD
            out_v[i, sl] = jnp.where(ok, xv + (i * KEYS_PER_FIELD), minus1)
        return carry

    lax.fori_loop(0, CVECS, group, 0)
    pltpu.sync_copy(out_v, out_hbm.at[:, pl.ds(base, BCOLS)])


@jax.jit
def kernel(x):
    run = pl.kernel(
        _body,
        out_type=jax.ShapeDtypeStruct((N_FIELDS, BATCH), jnp.int32),
        mesh=plsc.VectorSubcoreMesh(
            core_axis_name="c", subcore_axis_name="s",
            num_cores=NUM_CORES, num_subcores=NUM_SUBCORES,
        ),
        compiler_params=pltpu.CompilerParams(use_tc_tiling_on_sc=True),
        scratch_types=[
            pltpu.VMEM((N_FIELDS, BCOLS), jnp.int32),
            pltpu.VMEM((N_FIELDS, BCOLS), jnp.int32),
        ],
    )
    return run(x.T).T
